# baseline (speedup 1.0000x reference)
import functools

import jax
import jax.numpy as jnp
from jax import lax
from jax.experimental import pallas as pl
from jax.experimental.pallas import tpu as pltpu
from jax.experimental.pallas import tpu_sc as plsc

BB = 32
NPTS = 1024
KNN = 16
NCLS = 40

_NEG = -3.0e38
_INF = 3.0e38

NTOT = BB * NPTS
NW = 32
PW = NTOT // NW
SCP = 32
NSC = PW // SCP


def _knn_body(x0_ref, wa0_r, wb0_r, b0_r, idx_ref, a0_ref, c0p_ref):
    x0 = x0_ref[0]
    a0_ref[0] = jnp.dot(x0, wa0_r[...], preferred_element_type=jnp.float32) + b0_r[...]
    c0 = jnp.dot(x0, wb0_r[...], preferred_element_type=jnp.float32)
    c0p_ref[0] = jnp.concatenate([c0, jnp.zeros((NPTS, 64), jnp.float32)], axis=1)
    g = jax.lax.dot_general(x0, x0, (((1,), (1,)), ((), ())),
                            preferred_element_type=jnp.float32)
    r = jax.lax.broadcasted_iota(jnp.int32, (NPTS, NPTS), 0)
    c = jax.lax.broadcasted_iota(jnp.int32, (NPTS, NPTS), 1)
    eye = r == c
    x0sq = x0 * x0
    sq_r = jnp.sum(x0sq, axis=1, keepdims=True)
    ones14 = jnp.ones((1, 4), jnp.float32)
    sq_c = jax.lax.dot_general(ones14, x0sq, (((1,), (1,)), ((), ())),
                               preferred_element_type=jnp.float32)
    d2 = sq_r + sq_c - 2.0 * g
    d2m = jnp.where(eye, _INF, d2)
    off = pl.program_id(0) * NPTS
    ams = []
    for k in range(KNN):
        m = jnp.min(d2m, axis=1, keepdims=True)
        eq = d2m == m
        ams.append(jnp.min(jnp.where(eq, c, 1 << 30), axis=1, keepdims=True))
        if k + 1 < KNN:
            d2m = jnp.where(eq, _INF, d2m)
    idx_ref[0] = jnp.concatenate(ams, axis=1) + off


def _conv1_body(g0_ref, a0_ref, w1_r, b1_r, w2_r, b2_r,
                wa2_r, wb2_r, bb2_r, x1_ref, a2_ref, c2p_ref):
    relu = jax.nn.relu
    g = g0_ref[0][:, :64]
    a0 = a0_ref[0]
    a0t = jnp.broadcast_to(a0[:, None, :], (NPTS, KNN, 64)).reshape(KNN * NPTS, 64)
    h = relu(a0t + g)
    h = relu(jnp.dot(h, w1_r[...], preferred_element_type=jnp.float32) + b1_r[...])
    h = relu(jnp.dot(h, w2_r[...], preferred_element_type=jnp.float32) + b2_r[...])
    x1 = jnp.max(h.reshape(NPTS, KNN, 64), axis=1)

    x1_ref[0] = x1
    a2_ref[0] = jnp.dot(x1, wa2_r[...], preferred_element_type=jnp.float32) + bb2_r[...]
    c2 = jnp.dot(x1, wb2_r[...], preferred_element_type=jnp.float32)
    c2p_ref[0] = jnp.concatenate([c2, jnp.zeros((NPTS, 64), jnp.float32)], axis=1)


def _sc_gather_max(table, idx1d):
    mesh = plsc.VectorSubcoreMesh(core_axis_name="c", subcore_axis_name="s")

    @functools.partial(
        pl.kernel, mesh=mesh,
        out_type=jax.ShapeDtypeStruct((NTOT, 128), jnp.float32),
        scratch_types=[
            pltpu.VMEM((SCP * KNN,), jnp.int32),
            pltpu.VMEM((SCP * KNN, 128), jnp.float32),
            pltpu.VMEM((SCP, 128), jnp.float32),
            pltpu.SemaphoreType.DMA,
        ],
    )
    def k(tab_hbm, idx_hbm, out_hbm, idx_v, rows_v, out_v, sem):
        wid = lax.axis_index("s") * 2 + lax.axis_index("c")
        base_pt = wid * PW

        def body(i, _):
            pt0 = pl.multiple_of(base_pt + i * SCP, SCP)
            e0 = pl.multiple_of(pt0 * KNN, SCP * KNN)
            pltpu.sync_copy(idx_hbm.at[pl.ds(e0, SCP * KNN)], idx_v)
            cps = [
                pltpu.async_copy(
                    tab_hbm.at[idx_v.at[pl.ds(j * 128, 128)]],
                    rows_v.at[pl.ds(j * 128, 128)],
                    sem,
                )
                for j in range(SCP * KNN // 128)
            ]
            for cp in cps:
                cp.wait()

            def pbody(p, _):
                for ch in range(8):
                    acc = rows_v[p * KNN, pl.ds(ch * 16, 16)]
                    for kk in range(1, KNN):
                        acc = jnp.maximum(acc, rows_v[p * KNN + kk, pl.ds(ch * 16, 16)])
                    out_v[p, pl.ds(ch * 16, 16)] = acc
                return 0

            lax.fori_loop(0, SCP, pbody, 0)
            pltpu.sync_copy(out_v, out_hbm.at[pl.ds(pt0, SCP)])
            return 0

        lax.fori_loop(0, NSC, body, 0)

    return k(table, idx1d)


def _sc_gather_rows(table, idx1d):
    mesh = plsc.VectorSubcoreMesh(core_axis_name="c", subcore_axis_name="s")

    @functools.partial(
        pl.kernel, mesh=mesh,
        out_type=jax.ShapeDtypeStruct((NTOT * KNN, 128), jnp.float32),
        scratch_types=[
            pltpu.VMEM((SCP * KNN,), jnp.int32),
            pltpu.VMEM((SCP * KNN, 128), jnp.float32),
            pltpu.SemaphoreType.DMA,
        ],
    )
    def k(tab_hbm, idx_hbm, out_hbm, idx_v, rows_v, sem):
        wid = lax.axis_index("s") * 2 + lax.axis_index("c")
        base_pt = wid * PW

        def body(i, _):
            pt0 = pl.multiple_of(base_pt + i * SCP, SCP)
            e0 = pl.multiple_of(pt0 * KNN, SCP * KNN)
            pltpu.sync_copy(idx_hbm.at[pl.ds(e0, SCP * KNN)], idx_v)
            cps = [
                pltpu.async_copy(
                    tab_hbm.at[idx_v.at[pl.ds(j * 128, 128)]],
                    rows_v.at[pl.ds(j * 128, 128)],
                    sem,
                )
                for j in range(SCP * KNN // 128)
            ]
            for cp in cps:
                cp.wait()
            pltpu.sync_copy(rows_v, out_hbm.at[pl.ds(e0, SCP * KNN)])
            return 0

        lax.fori_loop(0, NSC, body, 0)

    return k(table, idx1d)


def _conv2_body(a2_ref, m2_ref, wa3_r, wb3_r, bb3_r, x2_ref, a3_ref, c3_ref):
    relu = jax.nn.relu
    x2 = relu(a2_ref[0] + m2_ref[0, :, :64])
    x2_ref[0] = x2
    a3_ref[0] = jnp.dot(x2, wa3_r[...], preferred_element_type=jnp.float32) + bb3_r[...]
    c3_ref[0] = jnp.dot(x2, wb3_r[...], preferred_element_type=jnp.float32)


def _lin1_body(x1_ref, x2_ref, a3_ref, m3_ref, l1w_r, l1b_r, out_ref):
    x3 = jax.nn.relu(a3_ref[0] + m3_ref[0])
    l1w = l1w_r[...]
    out1 = (jnp.dot(x1_ref[0], l1w[:64], preferred_element_type=jnp.float32)
            + jnp.dot(x2_ref[0], l1w[64:128], preferred_element_type=jnp.float32)
            + jnp.dot(x3, l1w[128:], preferred_element_type=jnp.float32)
            + l1b_r[...])
    out_ref[0] = jnp.max(out1, axis=0, keepdims=True)


def _cls_body(p_ref, w0_r, b0_r, w1_r, b1_r, w2_r, b2_r, out_ref):
    h = jax.nn.relu(jnp.dot(p_ref[...], w0_r[...],
                            preferred_element_type=jnp.float32) + b0_r[...])
    h = jax.nn.relu(jnp.dot(h, w1_r[...],
                            preferred_element_type=jnp.float32) + b1_r[...])
    logits = jnp.dot(h, w2_r[...], preferred_element_type=jnp.float32) + b2_r[...]
    mx = jnp.max(logits, axis=1, keepdims=True)
    s = logits - mx
    lse = jnp.log(jnp.sum(jnp.exp(s), axis=1, keepdims=True))
    out_ref[...] = s - lse


def _full(shape):
    return pl.BlockSpec(shape, lambda *_: (0,) * len(shape))


def _grid_spec(shape):
    return pl.BlockSpec((1,) + shape, lambda b: (b,) + (0,) * len(shape))


def kernel(pos, x, batch, c1_w0, c1_b0, c1_w1, c1_b1, c1_w2, c1_b2, c2_w0, c2_b0,
           c3_w0, c3_b0, lin1_w, lin1_b, m_w0, m_b0, m_w1, m_b1, m_w2, m_b2):
    x0 = jnp.concatenate([pos, x], axis=1).reshape(BB, NPTS, 4)

    wa0, wb0 = c1_w0[:4] - c1_w0[4:], c1_w0[4:]
    wa2, wb2 = c2_w0[:64] - c2_w0[64:], c2_w0[64:]
    wa3, wb3 = c3_w0[:64] - c3_w0[64:], c3_w0[64:]
    r2 = lambda v: v.reshape(1, -1)

    idx, a0, c0p = pl.pallas_call(
        _knn_body,
        grid=(BB,),
        in_specs=[_grid_spec((NPTS, 4)), _full(wa0.shape), _full(wb0.shape),
                  _full((1, 64))],
        out_specs=[_grid_spec((NPTS, KNN)), _grid_spec((NPTS, 64)),
                   _grid_spec((NPTS, 128))],
        out_shape=[jax.ShapeDtypeStruct((BB, NPTS, KNN), jnp.int32),
                   jax.ShapeDtypeStruct((BB, NPTS, 64), jnp.float32),
                   jax.ShapeDtypeStruct((BB, NPTS, 128), jnp.float32)],
    )(x0, wa0, wb0, r2(c1_b0))
    idx_flat = idx.reshape(NTOT * KNN)

    g0 = _sc_gather_rows(c0p.reshape(NTOT, 128), idx_flat)
    g0 = g0.reshape(BB, KNN * NPTS, 128)

    ws1 = [c1_w1, r2(c1_b1), c1_w2, r2(c1_b2), wa2, wb2, r2(c2_b0)]
    x1, a2, c2p = pl.pallas_call(
        _conv1_body,
        grid=(BB,),
        in_specs=[_grid_spec((KNN * NPTS, 128)), _grid_spec((NPTS, 64))]
                 + [_full(w.shape) for w in ws1],
        out_specs=[_grid_spec((NPTS, 64)), _grid_spec((NPTS, 64)),
                   _grid_spec((NPTS, 128))],
        out_shape=[jax.ShapeDtypeStruct((BB, NPTS, 64), jnp.float32),
                   jax.ShapeDtypeStruct((BB, NPTS, 64), jnp.float32),
                   jax.ShapeDtypeStruct((BB, NPTS, 128), jnp.float32)],
    )(g0, a0, *ws1)

    m2 = _sc_gather_max(c2p.reshape(NTOT, 128), idx_flat).reshape(BB, NPTS, 128)

    ws2 = [wa3, wb3, r2(c3_b0)]
    x2, a3, c3 = pl.pallas_call(
        _conv2_body,
        grid=(BB,),
        in_specs=[_grid_spec((NPTS, 64)), _grid_spec((NPTS, 128))]
                 + [_full(w.shape) for w in ws2],
        out_specs=[_grid_spec((NPTS, 64)), _grid_spec((NPTS, 128)),
                   _grid_spec((NPTS, 128))],
        out_shape=[jax.ShapeDtypeStruct((BB, NPTS, 64), jnp.float32),
                   jax.ShapeDtypeStruct((BB, NPTS, 128), jnp.float32),
                   jax.ShapeDtypeStruct((BB, NPTS, 128), jnp.float32)],
    )(a2, m2, *ws2)

    m3 = _sc_gather_max(c3.reshape(NTOT, 128), idx_flat).reshape(BB, NPTS, 128)

    pooled = pl.pallas_call(
        _lin1_body,
        grid=(BB,),
        in_specs=[_grid_spec((NPTS, 64)), _grid_spec((NPTS, 64)),
                  _grid_spec((NPTS, 128)), _grid_spec((NPTS, 128)),
                  _full(lin1_w.shape), _full((1, NPTS))],
        out_specs=pl.BlockSpec((1, 1, NPTS), lambda b: (b, 0, 0)),
        out_shape=jax.ShapeDtypeStruct((BB, 1, NPTS), jnp.float32),
    )(x1, x2, a3, m3, lin1_w, r2(lin1_b))
    pooled = pooled.reshape(BB, NPTS)

    cws = [m_w0, r2(m_b0), m_w1, r2(m_b1), m_w2, r2(m_b2)]
    return pl.pallas_call(
        _cls_body,
        in_specs=[_full((BB, NPTS))] + [_full(w.shape) for w in cws],
        out_specs=_full((BB, NCLS)),
        out_shape=jax.ShapeDtypeStruct((BB, NCLS), jnp.float32),
    )(pooled, *cws)


# speedup vs baseline: 1.0054x; 1.0054x over previous
import functools

import jax
import jax.numpy as jnp
from jax import lax
from jax.experimental import pallas as pl
from jax.experimental.pallas import tpu as pltpu
from jax.experimental.pallas import tpu_sc as plsc

BB = 32
NPTS = 1024
KNN = 16
NCLS = 40

_NEG = -3.0e38
_INF = 3.0e38

NTOT = BB * NPTS
NW = 32
PW = NTOT // NW
SCP = 32
NSC = PW // SCP


def _knn_body(x0_ref, wa0_r, wb0_r, b0_r, idx_ref, a0_ref, c0p_ref):
    x0 = x0_ref[0]
    a0_ref[0] = jnp.dot(x0, wa0_r[...], preferred_element_type=jnp.float32) + b0_r[...]
    c0 = jnp.dot(x0, wb0_r[...], preferred_element_type=jnp.float32)
    c0p_ref[0] = jnp.concatenate([c0, jnp.zeros((NPTS, 64), jnp.float32)], axis=1)
    g = jax.lax.dot_general(x0, x0, (((1,), (1,)), ((), ())),
                            preferred_element_type=jnp.float32)
    r = jax.lax.broadcasted_iota(jnp.int32, (NPTS, NPTS), 0)
    c = jax.lax.broadcasted_iota(jnp.int32, (NPTS, NPTS), 1)
    eye = r == c
    diag = jnp.where(eye, g, 0.0)
    sq_r = jnp.sum(diag, axis=1, keepdims=True)
    sq_c = jnp.sum(diag, axis=0, keepdims=True)
    d2 = sq_r + sq_c - 2.0 * g
    d2m = jnp.where(eye, _INF, d2)
    off = pl.program_id(0) * NPTS
    for k in range(KNN):
        m = jnp.min(d2m, axis=1, keepdims=True)
        eq = d2m == m
        am = jnp.min(jnp.where(eq, c, 1 << 30), axis=1, keepdims=True)
        idx_ref[0, :, k:k + 1] = am + off
        if k + 1 < KNN:
            d2m = jnp.where(eq, _INF, d2m)


def _conv1_body(g0_ref, a0_ref, w1_r, b1_r, w2_r, b2_r,
                wa2_r, wb2_r, bb2_r, x1_ref, a2_ref, c2p_ref):
    relu = jax.nn.relu
    g = g0_ref[0][:, :64]
    a0 = a0_ref[0]
    a0t = jnp.broadcast_to(a0[:, None, :], (NPTS, KNN, 64)).reshape(KNN * NPTS, 64)
    h = relu(a0t + g)
    h = relu(jnp.dot(h, w1_r[...], preferred_element_type=jnp.float32) + b1_r[...])
    h = relu(jnp.dot(h, w2_r[...], preferred_element_type=jnp.float32) + b2_r[...])
    x1 = jnp.max(h.reshape(NPTS, KNN, 64), axis=1)

    x1_ref[0] = x1
    a2_ref[0] = jnp.dot(x1, wa2_r[...], preferred_element_type=jnp.float32) + bb2_r[...]
    c2 = jnp.dot(x1, wb2_r[...], preferred_element_type=jnp.float32)
    c2p_ref[0] = jnp.concatenate([c2, jnp.zeros((NPTS, 64), jnp.float32)], axis=1)


def _sc_gather_max(table, idx1d):
    mesh = plsc.VectorSubcoreMesh(core_axis_name="c", subcore_axis_name="s")

    @functools.partial(
        pl.kernel, mesh=mesh,
        out_type=jax.ShapeDtypeStruct((NTOT, 128), jnp.float32),
        scratch_types=[
            pltpu.VMEM((SCP * KNN,), jnp.int32),
            pltpu.VMEM((SCP * KNN, 128), jnp.float32),
            pltpu.VMEM((SCP, 128), jnp.float32),
            pltpu.SemaphoreType.DMA,
        ],
    )
    def k(tab_hbm, idx_hbm, out_hbm, idx_v, rows_v, out_v, sem):
        wid = lax.axis_index("s") * 2 + lax.axis_index("c")
        base_pt = wid * PW

        def body(i, _):
            pt0 = pl.multiple_of(base_pt + i * SCP, SCP)
            e0 = pl.multiple_of(pt0 * KNN, SCP * KNN)
            pltpu.sync_copy(idx_hbm.at[pl.ds(e0, SCP * KNN)], idx_v)
            cps = [
                pltpu.async_copy(
                    tab_hbm.at[idx_v.at[pl.ds(j * 128, 128)]],
                    rows_v.at[pl.ds(j * 128, 128)],
                    sem,
                )
                for j in range(SCP * KNN // 128)
            ]
            for cp in cps:
                cp.wait()

            def pbody(p, _):
                for ch in range(8):
                    acc = rows_v[p * KNN, pl.ds(ch * 16, 16)]
                    for kk in range(1, KNN):
                        acc = jnp.maximum(acc, rows_v[p * KNN + kk, pl.ds(ch * 16, 16)])
                    out_v[p, pl.ds(ch * 16, 16)] = acc
                return 0

            lax.fori_loop(0, SCP, pbody, 0)
            pltpu.sync_copy(out_v, out_hbm.at[pl.ds(pt0, SCP)])
            return 0

        lax.fori_loop(0, NSC, body, 0)

    return k(table, idx1d)


def _sc_gather_rows(table, idx1d):
    mesh = plsc.VectorSubcoreMesh(core_axis_name="c", subcore_axis_name="s")

    @functools.partial(
        pl.kernel, mesh=mesh,
        out_type=jax.ShapeDtypeStruct((NTOT * KNN, 128), jnp.float32),
        scratch_types=[
            pltpu.VMEM((SCP * KNN,), jnp.int32),
            pltpu.VMEM((SCP * KNN, 128), jnp.float32),
            pltpu.SemaphoreType.DMA,
        ],
    )
    def k(tab_hbm, idx_hbm, out_hbm, idx_v, rows_v, sem):
        wid = lax.axis_index("s") * 2 + lax.axis_index("c")
        base_pt = wid * PW

        def body(i, _):
            pt0 = pl.multiple_of(base_pt + i * SCP, SCP)
            e0 = pl.multiple_of(pt0 * KNN, SCP * KNN)
            pltpu.sync_copy(idx_hbm.at[pl.ds(e0, SCP * KNN)], idx_v)
            cps = [
                pltpu.async_copy(
                    tab_hbm.at[idx_v.at[pl.ds(j * 128, 128)]],
                    rows_v.at[pl.ds(j * 128, 128)],
                    sem,
                )
                for j in range(SCP * KNN // 128)
            ]
            for cp in cps:
                cp.wait()
            pltpu.sync_copy(rows_v, out_hbm.at[pl.ds(e0, SCP * KNN)])
            return 0

        lax.fori_loop(0, NSC, body, 0)

    return k(table, idx1d)


def _conv2_body(a2_ref, m2_ref, wa3_r, wb3_r, bb3_r, x2_ref, a3_ref, c3_ref):
    relu = jax.nn.relu
    x2 = relu(a2_ref[0] + m2_ref[0, :, :64])
    x2_ref[0] = x2
    a3_ref[0] = jnp.dot(x2, wa3_r[...], preferred_element_type=jnp.float32) + bb3_r[...]
    c3_ref[0] = jnp.dot(x2, wb3_r[...], preferred_element_type=jnp.float32)


def _lin1_body(x1_ref, x2_ref, a3_ref, m3_ref, l1w_r, l1b_r, out_ref):
    x3 = jax.nn.relu(a3_ref[0] + m3_ref[0])
    l1w = l1w_r[...]
    out1 = (jnp.dot(x1_ref[0], l1w[:64], preferred_element_type=jnp.float32)
            + jnp.dot(x2_ref[0], l1w[64:128], preferred_element_type=jnp.float32)
            + jnp.dot(x3, l1w[128:], preferred_element_type=jnp.float32)
            + l1b_r[...])
    out_ref[0] = jnp.max(out1, axis=0, keepdims=True)


def _cls_body(p_ref, w0_r, b0_r, w1_r, b1_r, w2_r, b2_r, out_ref):
    h = jax.nn.relu(jnp.dot(p_ref[...], w0_r[...],
                            preferred_element_type=jnp.float32) + b0_r[...])
    h = jax.nn.relu(jnp.dot(h, w1_r[...],
                            preferred_element_type=jnp.float32) + b1_r[...])
    logits = jnp.dot(h, w2_r[...], preferred_element_type=jnp.float32) + b2_r[...]
    mx = jnp.max(logits, axis=1, keepdims=True)
    s = logits - mx
    lse = jnp.log(jnp.sum(jnp.exp(s), axis=1, keepdims=True))
    out_ref[...] = s - lse


def _full(shape):
    return pl.BlockSpec(shape, lambda *_: (0,) * len(shape))


def _grid_spec(shape):
    return pl.BlockSpec((1,) + shape, lambda b: (b,) + (0,) * len(shape))


def kernel(pos, x, batch, c1_w0, c1_b0, c1_w1, c1_b1, c1_w2, c1_b2, c2_w0, c2_b0,
           c3_w0, c3_b0, lin1_w, lin1_b, m_w0, m_b0, m_w1, m_b1, m_w2, m_b2):
    x0 = jnp.concatenate([pos, x], axis=1).reshape(BB, NPTS, 4)

    wa0, wb0 = c1_w0[:4] - c1_w0[4:], c1_w0[4:]
    wa2, wb2 = c2_w0[:64] - c2_w0[64:], c2_w0[64:]
    wa3, wb3 = c3_w0[:64] - c3_w0[64:], c3_w0[64:]
    r2 = lambda v: v.reshape(1, -1)

    idx, a0, c0p = pl.pallas_call(
        _knn_body,
        grid=(BB,),
        in_specs=[_grid_spec((NPTS, 4)), _full(wa0.shape), _full(wb0.shape),
                  _full((1, 64))],
        out_specs=[_grid_spec((NPTS, KNN)), _grid_spec((NPTS, 64)),
                   _grid_spec((NPTS, 128))],
        out_shape=[jax.ShapeDtypeStruct((BB, NPTS, KNN), jnp.int32),
                   jax.ShapeDtypeStruct((BB, NPTS, 64), jnp.float32),
                   jax.ShapeDtypeStruct((BB, NPTS, 128), jnp.float32)],
    )(x0, wa0, wb0, r2(c1_b0))
    idx_flat = idx.reshape(NTOT * KNN)

    g0 = _sc_gather_rows(c0p.reshape(NTOT, 128), idx_flat)
    g0 = g0.reshape(BB, KNN * NPTS, 128)

    ws1 = [c1_w1, r2(c1_b1), c1_w2, r2(c1_b2), wa2, wb2, r2(c2_b0)]
    x1, a2, c2p = pl.pallas_call(
        _conv1_body,
        grid=(BB,),
        in_specs=[_grid_spec((KNN * NPTS, 128)), _grid_spec((NPTS, 64))]
                 + [_full(w.shape) for w in ws1],
        out_specs=[_grid_spec((NPTS, 64)), _grid_spec((NPTS, 64)),
                   _grid_spec((NPTS, 128))],
        out_shape=[jax.ShapeDtypeStruct((BB, NPTS, 64), jnp.float32),
                   jax.ShapeDtypeStruct((BB, NPTS, 64), jnp.float32),
                   jax.ShapeDtypeStruct((BB, NPTS, 128), jnp.float32)],
    )(g0, a0, *ws1)

    m2 = _sc_gather_max(c2p.reshape(NTOT, 128), idx_flat).reshape(BB, NPTS, 128)

    ws2 = [wa3, wb3, r2(c3_b0)]
    x2, a3, c3 = pl.pallas_call(
        _conv2_body,
        grid=(BB,),
        in_specs=[_grid_spec((NPTS, 64)), _grid_spec((NPTS, 128))]
                 + [_full(w.shape) for w in ws2],
        out_specs=[_grid_spec((NPTS, 64)), _grid_spec((NPTS, 128)),
                   _grid_spec((NPTS, 128))],
        out_shape=[jax.ShapeDtypeStruct((BB, NPTS, 64), jnp.float32),
                   jax.ShapeDtypeStruct((BB, NPTS, 128), jnp.float32),
                   jax.ShapeDtypeStruct((BB, NPTS, 128), jnp.float32)],
    )(a2, m2, *ws2)

    m3 = _sc_gather_max(c3.reshape(NTOT, 128), idx_flat).reshape(BB, NPTS, 128)

    pooled = pl.pallas_call(
        _lin1_body,
        grid=(BB,),
        in_specs=[_grid_spec((NPTS, 64)), _grid_spec((NPTS, 64)),
                  _grid_spec((NPTS, 128)), _grid_spec((NPTS, 128)),
                  _full(lin1_w.shape), _full((1, NPTS))],
        out_specs=pl.BlockSpec((1, 1, NPTS), lambda b: (b, 0, 0)),
        out_shape=jax.ShapeDtypeStruct((BB, 1, NPTS), jnp.float32),
    )(x1, x2, a3, m3, lin1_w, r2(lin1_b))
    pooled = pooled.reshape(BB, NPTS)

    cws = [m_w0, r2(m_b0), m_w1, r2(m_b1), m_w2, r2(m_b2)]
    return pl.pallas_call(
        _cls_body,
        in_specs=[_full((BB, NPTS))] + [_full(w.shape) for w in cws],
        out_specs=_full((BB, NCLS)),
        out_shape=jax.ShapeDtypeStruct((BB, NCLS), jnp.float32),
    )(pooled, *cws)
